# Initial kernel scaffold; baseline (speedup 1.0000x reference)
import jax
import jax.numpy as jnp
from jax.experimental import pallas as pl


def kernel(inpt, masks, labels):
    raise NotImplementedError("write your pallas kernel here")


# speedup vs baseline: 3.6084x; 3.6084x over previous
import jax
import jax.numpy as jnp
from jax.experimental import pallas as pl
from jax.experimental.pallas import tpu as pltpu


def _winner_weights(labels, n, dtype):
    lab = labels.astype(jnp.int32)
    idx = jnp.arange(n, dtype=jnp.int32)
    later_dup = (lab[None, :] == lab[:, None]) & (idx[None, :] > idx[:, None])
    return (~later_dup.any(axis=1)).astype(dtype)


def _body(w_ref, m_ref, o_ref):
    i = pl.program_id(0)

    @pl.when(i == 0)
    def _():
        o_ref[...] = jnp.zeros_like(o_ref)

    o_ref[...] += w_ref[i] * m_ref[0]


def kernel(inpt, masks, labels):
    n, h, w = masks.shape
    wts = _winner_weights(labels, n, masks.dtype)

    std_mask = pl.pallas_call(
        _body,
        grid=(n,),
        in_specs=[
            pl.BlockSpec(memory_space=pltpu.SMEM),
            pl.BlockSpec((1, h, w), lambda i: (i, 0, 0)),
        ],
        out_specs=pl.BlockSpec((h, w), lambda i: (0, 0)),
        out_shape=jax.ShapeDtypeStruct((h, w), masks.dtype),
    )(wts, masks)
    return (inpt, std_mask)
